# baseline (speedup 1.0000x reference)
import functools

import jax
import jax.numpy as jnp
from jax import lax
from jax.experimental import pallas as pl
from jax.experimental.pallas import tpu as pltpu
from jax.experimental.pallas import tpu_sc as plsc

_N_NODES = 10000
_N_EDGES = 320000
_D = 128

_NC = 2
_NS = 16
_NW = _NC * _NS

_EPT = _N_EDGES // _NW
_CHUNK = 128
_NBUF = 3
_FULL_CHUNKS = _EPT // _CHUNK
_REM = _EPT - _FULL_CHUNKS * _CHUNK
_RING_ITERS = _FULL_CHUNKS // _NBUF

_N_PAD = 10112
_ROWS_PER_TILE = _N_PAD // _NS
_ZROWS = 128


def _sc_scatter_add(gathered_hbm, dst_hbm, out_hbm, acc,
                    f0, f1, f2, i0, i1, i2, idx16_v,
                    sf0, sf1, sf2, si0, si1, si2):
  feats = (f0, f1, f2)
  idxs = (i0, i1, i2)
  sem_f = (sf0, sf1, sf2)
  sem_i = (si0, si1, si2)

  c = lax.axis_index("c")
  s = lax.axis_index("s")
  w = c * _NS + s
  base = w * _EPT

  def start_dma(b, k):
    e0 = pl.multiple_of(base + k * _CHUNK, 16)
    pltpu.async_copy(dst_hbm.at[pl.ds(e0, _CHUNK)], idxs[b], sem_i[b])
    pltpu.async_copy(gathered_hbm.at[pl.ds(e0, _CHUNK), 0], feats[b], sem_f[b])

  start_dma(1, 1)
  start_dma(2, 2)

  z = jnp.zeros((16,), jnp.float32)

  def zero_row(r, carry):
    for l in range(_D // 16):
      f0[r, pl.ds(16 * l, 16)] = z
    return carry

  lax.fori_loop(0, _ZROWS, zero_row, 0)
  row0 = s * _ROWS_PER_TILE
  for r in range(0, _ROWS_PER_TILE, _ZROWS):
    n = min(_ZROWS, _ROWS_PER_TILE - r)
    pltpu.sync_copy(f0.at[pl.ds(0, n)], acc.at[pl.ds(row0 + r, n)])
  start_dma(0, 0)
  plsc.subcore_barrier()

  def ring(j, carry):
    for t in range(_NBUF):
      k = _NBUF * j + t

      @pl.when(k + 2 < _FULL_CHUNKS)
      def _(t=t, k=k):
        start_dma((t + 2) % _NBUF, k + 2)

      e0 = pl.multiple_of(base + k * _CHUNK, 16)
      pltpu.make_async_copy(dst_hbm.at[pl.ds(e0, _CHUNK)], idxs[t],
                            sem_i[t]).wait()
      pltpu.make_async_copy(gathered_hbm.at[pl.ds(e0, _CHUNK), 0], feats[t],
                            sem_f[t]).wait()
      pltpu.sync_copy(feats[t], acc.at[idxs[t]], add=True)
    return carry

  lax.fori_loop(0, _RING_ITERS, ring, 0)

  e0 = pl.multiple_of(base + _FULL_CHUNKS * _CHUNK, 8)
  pltpu.sync_copy(dst_hbm.at[pl.ds(e0, _REM)], idx16_v)
  pltpu.sync_copy(gathered_hbm.at[pl.ds(e0, _REM), 0], f0.at[pl.ds(0, _REM)])
  pltpu.sync_copy(f0.at[pl.ds(0, _REM)], acc.at[idx16_v], add=True)

  plsc.subcore_barrier()

  for r in range(0, _ROWS_PER_TILE, _ZROWS):
    n = min(_ZROWS, _ROWS_PER_TILE - r)
    pltpu.sync_copy(acc.at[pl.ds(row0 + r, n)],
                    out_hbm.at[c, pl.ds(row0 + r, n)])


def _combine_body(p_ref, o_ref):
  o_ref[...] = p_ref[0] + p_ref[1]


def kernel(node_features, gathered_nodes, edge_list):
  del node_features
  dst = edge_list[:, 1]

  mesh = plsc.VectorSubcoreMesh(core_axis_name="c", subcore_axis_name="s")
  sc = pl.kernel(
      _sc_scatter_add,
      out_type=jax.ShapeDtypeStruct((_NC, _N_PAD, _D), jnp.float32),
      mesh=mesh,
      scratch_types=[
          pltpu.VMEM_SHARED((_N_PAD, _D), jnp.float32),
      ] + [pltpu.VMEM((_CHUNK, _D), jnp.float32)] * _NBUF
        + [pltpu.VMEM((_CHUNK,), jnp.int32)] * _NBUF
        + [pltpu.VMEM((_REM,), jnp.int32)]
        + [pltpu.SemaphoreType.DMA] * (2 * _NBUF),
  )
  partials = sc(gathered_nodes, dst)

  rows_blk = 1000
  out = pl.pallas_call(
      _combine_body,
      out_shape=jax.ShapeDtypeStruct((_N_NODES, _D), jnp.float32),
      grid=(_N_NODES // rows_blk,),
      in_specs=[pl.BlockSpec((_NC, rows_blk, _D), lambda i: (0, i, 0))],
      out_specs=pl.BlockSpec((rows_blk, _D), lambda i: (i, 0)),
  )(partials)
  return out


# speedup vs baseline: 1.0124x; 1.0124x over previous
import functools

import jax
import jax.numpy as jnp
from jax import lax
from jax.experimental import pallas as pl
from jax.experimental.pallas import tpu as pltpu
from jax.experimental.pallas import tpu_sc as plsc

_N_NODES = 10000
_N_EDGES = 320000
_D = 128

_NC = 2
_NS = 16
_NW = _NC * _NS

_EPT = _N_EDGES // _NW
_CHUNK = 128
_NBUF = 3
_FULL_CHUNKS = _EPT // _CHUNK
_REM = _EPT - _FULL_CHUNKS * _CHUNK
_RING_ITERS = _FULL_CHUNKS // _NBUF

_N_PAD = 10112
_ROWS_PER_TILE = _N_PAD // _NS
_ZROWS = 128


def _sc_scatter_add(gathered_hbm, dst_hbm, out_hbm, acc,
                    f0, f1, f2, i0, i1, i2, idx16_v,
                    sf0, sf1, sf2, si0, si1, si2):
  feats = (f0, f1, f2)
  idxs = (i0, i1, i2)
  sem_f = (sf0, sf1, sf2)
  sem_i = (si0, si1, si2)

  c = lax.axis_index("c")
  s = lax.axis_index("s")
  w = c * _NS + s
  base = w * _EPT

  def start_dma(b, k):
    e0 = pl.multiple_of(base + k * _CHUNK, 16)
    pltpu.async_copy(dst_hbm.at[pl.ds(e0, _CHUNK)], idxs[b], sem_i[b])
    pltpu.async_copy(gathered_hbm.at[pl.ds(e0, _CHUNK), 0], feats[b], sem_f[b])

  start_dma(1, 1)
  start_dma(2, 2)

  z = jnp.zeros((16,), jnp.float32)

  def zero_row(r, carry):
    for l in range(_D // 16):
      f0[r, pl.ds(16 * l, 16)] = z
    return carry

  lax.fori_loop(0, _ZROWS, zero_row, 0)
  row0 = s * _ROWS_PER_TILE
  for r in range(0, _ROWS_PER_TILE, _ZROWS):
    n = min(_ZROWS, _ROWS_PER_TILE - r)
    pltpu.sync_copy(f0.at[pl.ds(0, n)], acc.at[pl.ds(row0 + r, n)])
  start_dma(0, 0)
  plsc.subcore_barrier()

  def ring(j, carry):
    for t in range(_NBUF):
      k = _NBUF * j + t
      e0 = pl.multiple_of(base + k * _CHUNK, 16)
      pltpu.make_async_copy(dst_hbm.at[pl.ds(e0, _CHUNK)], idxs[t],
                            sem_i[t]).wait()
      pltpu.make_async_copy(gathered_hbm.at[pl.ds(e0, _CHUNK), 0], feats[t],
                            sem_f[t]).wait()
      pltpu.sync_copy(feats[t], acc.at[idxs[t]], add=True)

      @pl.when(k + _NBUF < _FULL_CHUNKS)
      def _(t=t, k=k):
        start_dma(t, k + _NBUF)
    return carry

  lax.fori_loop(0, _RING_ITERS, ring, 0)

  e0 = pl.multiple_of(base + _FULL_CHUNKS * _CHUNK, 8)
  pltpu.sync_copy(dst_hbm.at[pl.ds(e0, _REM)], idx16_v)
  pltpu.sync_copy(gathered_hbm.at[pl.ds(e0, _REM), 0], f0.at[pl.ds(0, _REM)])
  pltpu.sync_copy(f0.at[pl.ds(0, _REM)], acc.at[idx16_v], add=True)

  plsc.subcore_barrier()

  for r in range(0, _ROWS_PER_TILE, _ZROWS):
    n = min(_ZROWS, _ROWS_PER_TILE - r)
    pltpu.sync_copy(acc.at[pl.ds(row0 + r, n)],
                    out_hbm.at[c, pl.ds(row0 + r, n)])


def _combine_body(p_ref, o_ref):
  o_ref[...] = p_ref[0] + p_ref[1]


def kernel(node_features, gathered_nodes, edge_list):
  del node_features
  dst = edge_list[:, 1]

  mesh = plsc.VectorSubcoreMesh(core_axis_name="c", subcore_axis_name="s")
  sc = pl.kernel(
      _sc_scatter_add,
      out_type=jax.ShapeDtypeStruct((_NC, _N_PAD, _D), jnp.float32),
      mesh=mesh,
      scratch_types=[
          pltpu.VMEM_SHARED((_N_PAD, _D), jnp.float32),
      ] + [pltpu.VMEM((_CHUNK, _D), jnp.float32)] * _NBUF
        + [pltpu.VMEM((_CHUNK,), jnp.int32)] * _NBUF
        + [pltpu.VMEM((_REM,), jnp.int32)]
        + [pltpu.SemaphoreType.DMA] * (2 * _NBUF),
  )
  partials = sc(gathered_nodes, dst)

  rows_blk = 1000
  out = pl.pallas_call(
      _combine_body,
      out_shape=jax.ShapeDtypeStruct((_N_NODES, _D), jnp.float32),
      grid=(_N_NODES // rows_blk,),
      in_specs=[pl.BlockSpec((_NC, rows_blk, _D), lambda i: (0, i, 0))],
      out_specs=pl.BlockSpec((rows_blk, _D), lambda i: (i, 0)),
  )(partials)
  return out
